# baseline (speedup 1.0000x reference)
import functools

import jax
import jax.numpy as jnp
from jax import lax
from jax.experimental import pallas as pl
from jax.experimental.pallas import tpu as pltpu
from jax.experimental.pallas import tpu_sc as plsc

N = 10000
F = 128
NC, NS = 2, 16
NW = NC * NS
CHUNK = 128
NPAD = N + 112
DEGP = 10240


def _tile_base(c, s, cpt0, cpt1):
    return jnp.where(c == 0, s * cpt0, NS * cpt0 + s * cpt1)


def _deg_body(dstc_hbm, zeros_hbm, out_hbm, dst_v, ones_v, acc_sh, cpt0, cpt1):
    c = lax.axis_index("c")
    s = lax.axis_index("s")
    seg = DEGP // NS
    base = _tile_base(c, s, cpt0, cpt1)
    cptc = jnp.where(c == 0, cpt0, cpt1)
    pltpu.sync_copy(zeros_hbm.at[pl.ds(s * seg, seg)],
                    acc_sh.at[pl.ds(s * seg, seg)])
    pltpu.sync_copy(dstc_hbm.at[pl.ds(base, max(cpt0, cpt1))], dst_v)
    for i in range(CHUNK // 16):
        ones_v[pl.ds(i * 16, 16)] = jnp.ones((16,), jnp.float32)
    plsc.subcore_barrier()

    @pl.loop(0, cptc)
    def _(j):
        pltpu.sync_copy(ones_v, acc_sh.at[dst_v.at[j]], add=True)

    plsc.subcore_barrier()
    pltpu.sync_copy(acc_sh.at[pl.ds(s * seg, seg)],
                    out_hbm.at[c, pl.ds(s * seg, seg)])


def _agg_body(u_hbm, srcc_hbm, dstc_hbm, zeros_hbm, out_hbm,
              sidx, didx, rows_v, acc_sh, sl, sg0, sg1, ss0, ss1, cpt0, cpt1):
    c = lax.axis_index("c")
    s = lax.axis_index("s")
    seg = NPAD // NS
    base = _tile_base(c, s, cpt0, cpt1)
    GRP = 8
    ngroups = jnp.where(c == 0, cpt0 // GRP, cpt1 // GRP)
    pltpu.sync_copy(zeros_hbm.at[pl.ds(s * seg, seg)],
                    acc_sh.at[pl.ds(s * seg, seg)])
    pltpu.sync_copy(srcc_hbm.at[pl.ds(base, GRP)], sidx.at[0])
    pltpu.sync_copy(dstc_hbm.at[pl.ds(base, GRP)], didx.at[0])
    plsc.subcore_barrier()

    sg = (sg0, sg1)
    ss = (ss0, ss1)
    wtg = tuple(pltpu.make_async_copy(u_hbm.at[sidx.at[0, 0]],
                                      rows_v.at[b], sg[b]) for b in range(2))
    wts = tuple(pltpu.make_async_copy(rows_v.at[b],
                                      acc_sh.at[didx.at[0, 0]], ss[b])
                for b in range(2))
    wtl = (pltpu.make_async_copy(srcc_hbm.at[pl.ds(0, GRP)], sidx.at[0], sl),
           pltpu.make_async_copy(dstc_hbm.at[pl.ds(0, GRP)], didx.at[0], sl))

    def gath(p, r, b):
        pltpu.async_copy(u_hbm.at[sidx.at[p, r]], rows_v.at[b], sg[b])

    def scat(p, r, b):
        pltpu.async_copy(rows_v.at[b], acc_sh.at[didx.at[p, r]], ss[b],
                         add=True)

    gath(0, 0, 0)

    def do_group(g, p, pn, first):
        for r in range(GRP):
            b = r % 2
            wtg[b].wait()
            scat(p, r, b)
            if r == 0:
                nb = base + (g + 1) * GRP
                pltpu.async_copy(srcc_hbm.at[pl.ds(nb, GRP)], sidx.at[pn], sl)
                pltpu.async_copy(dstc_hbm.at[pl.ds(nb, GRP)], didx.at[pn], sl)
            if not (first and r == 0):
                wts[1 - b].wait()
            if r < GRP - 1:
                gath(p, r + 1, 1 - b)
            else:
                wtl[0].wait()
                wtl[1].wait()
                gath(pn, 0, 1 - b)

    do_group(0, 0, 1, True)

    @pl.loop(1, ngroups)
    def _(g):
        p = lax.rem(g, 2)
        do_group(g, p, 1 - p, False)

    wts[1].wait()
    wtg[0].wait()
    plsc.subcore_barrier()
    pltpu.sync_copy(acc_sh.at[pl.ds(s * seg, seg)],
                    out_hbm.at[c, pl.ds(s * seg, seg)])


def _sc_mesh():
    return plsc.VectorSubcoreMesh(core_axis_name="c", subcore_axis_name="s",
                                  num_cores=NC, num_subcores=NS)


def _deg_call(dstc, zeros1, cpt0, cpt1):
    return pl.kernel(
        functools.partial(_deg_body, cpt0=cpt0, cpt1=cpt1),
        out_type=jax.ShapeDtypeStruct((NC, DEGP), jnp.float32),
        mesh=_sc_mesh(),
        scratch_types=[
            pltpu.VMEM((max(cpt0, cpt1), CHUNK), jnp.int32),
            pltpu.VMEM((CHUNK,), jnp.float32),
            pltpu.VMEM_SHARED((DEGP,), jnp.float32),
        ],
    )(dstc, zeros1)


def _agg_call(u, srcc, dstc, zeros2, cpt0, cpt1):
    return pl.kernel(
        functools.partial(_agg_body, cpt0=cpt0, cpt1=cpt1),
        out_type=jax.ShapeDtypeStruct((NC, NPAD, F), jnp.float32),
        mesh=_sc_mesh(),
        scratch_types=[
            pltpu.VMEM((2, 8, CHUNK), jnp.int32),
            pltpu.VMEM((2, 8, CHUNK), jnp.int32),
            pltpu.VMEM((2, CHUNK, F), jnp.float32),
            pltpu.VMEM_SHARED((NPAD, F), jnp.float32),
            pltpu.SemaphoreType.DMA,
            pltpu.SemaphoreType.DMA,
            pltpu.SemaphoreType.DMA,
            pltpu.SemaphoreType.DMA,
            pltpu.SemaphoreType.DMA,
        ],
    )(u, srcc, dstc, zeros2)


def _tc1_body(x_ref, degp_ref, u1_ref, dis_ref):
    deg = degp_ref[0] + degp_ref[1] + 1.0
    dis = lax.rsqrt(deg)
    dis_ref[...] = dis
    u1_ref[...] = x_ref[...] * dis


def _tc2_body(p_ref, u1_ref, dis_ref, w1_ref, b1_ref, w2_ref, u2_ref):
    dis = dis_ref[...]
    agg1 = (p_ref[0] + p_ref[1] + u1_ref[...]) * dis
    h1 = jax.nn.relu(
        jnp.dot(agg1, w1_ref[...], preferred_element_type=jnp.float32)
        + b1_ref[...])
    g2 = jnp.dot(h1, w2_ref[...], preferred_element_type=jnp.float32)
    u2_ref[...] = g2 * dis


def _tc3_body(q_ref, u2_ref, dis_ref, b2_ref, wl_ref, bl_ref, out_ref):
    agg2 = (q_ref[0] + q_ref[1] + u2_ref[...]) * dis_ref[...]
    h2 = jax.nn.relu(agg2 + b2_ref[...])
    lg = jnp.dot(h2, wl_ref[...], preferred_element_type=jnp.float32) + bl_ref[...]
    m = jnp.max(lg, axis=-1, keepdims=True)
    lse = m + jnp.log(jnp.sum(jnp.exp(lg - m), axis=-1, keepdims=True))
    out_ref[...] = lg - lse


_BN = 2000


def kernel(x, edge_index, W1, b1, W2, b2, Wl, bl):
    E = edge_index.shape[1]
    pair = -(-E // (NS * CHUNK))
    pair = -(-pair // 16) * 16
    cpt1 = pair // 2
    cpt0 = pair - cpt1
    tot = NS * pair * CHUNK
    src = edge_index[0]
    dst = edge_index[1]
    pad = tot - E
    n0 = NS * cpt0 * CHUNK
    padrows = max(cpt0 - cpt1, 8)
    srcp = jnp.concatenate(
        [src[:n0],
         src[n0:] + N,
         jnp.full((pad,), N, jnp.int32),
         jnp.zeros((padrows * CHUNK,), jnp.int32)])
    srcc = srcp.reshape(NS * pair + padrows, CHUNK)
    dstc = jnp.concatenate(
        [dst, jnp.full((pad,), N, jnp.int32),
         jnp.zeros((padrows * CHUNK,), jnp.int32)]).reshape(
        NS * pair + padrows, CHUNK)
    zeros1 = jnp.zeros((DEGP,), jnp.float32)
    zeros2 = jnp.zeros((NPAD, F), jnp.float32)

    degp = _deg_call(dstc, zeros1, cpt0, cpt1)
    degp2 = degp[:, :, None]

    grid = (N // _BN,)
    row3 = lambda i: (0, i, 0)
    row2 = lambda i: (i, 0)
    whole = lambda i: (0, 0)

    u1, dis = pl.pallas_call(
        _tc1_body,
        grid=grid,
        in_specs=[
            pl.BlockSpec((_BN, F), row2),
            pl.BlockSpec((NC, _BN, 1), row3),
        ],
        out_specs=[
            pl.BlockSpec((_BN, F), row2),
            pl.BlockSpec((_BN, 1), row2),
        ],
        out_shape=[
            jax.ShapeDtypeStruct((N, F), jnp.float32),
            jax.ShapeDtypeStruct((N, 1), jnp.float32),
        ],
    )(x, degp2)

    p = _agg_call(jnp.concatenate([u1, u1]), srcc, dstc, zeros2, cpt0, cpt1)

    u2 = pl.pallas_call(
        _tc2_body,
        grid=grid,
        in_specs=[
            pl.BlockSpec((NC, _BN, F), row3),
            pl.BlockSpec((_BN, F), row2),
            pl.BlockSpec((_BN, 1), row2),
            pl.BlockSpec((F, 2 * F), whole),
            pl.BlockSpec((1, 2 * F), whole),
            pl.BlockSpec((2 * F, F), whole),
        ],
        out_specs=pl.BlockSpec((_BN, F), row2),
        out_shape=jax.ShapeDtypeStruct((N, F), jnp.float32),
    )(p, u1, dis, W1, b1.reshape(1, -1), W2)

    q = _agg_call(jnp.concatenate([u2, u2]), srcc, dstc, zeros2, cpt0, cpt1)

    C = Wl.shape[1]
    out = pl.pallas_call(
        _tc3_body,
        grid=grid,
        in_specs=[
            pl.BlockSpec((NC, _BN, F), row3),
            pl.BlockSpec((_BN, F), row2),
            pl.BlockSpec((_BN, 1), row2),
            pl.BlockSpec((1, F), whole),
            pl.BlockSpec((F, C), whole),
            pl.BlockSpec((1, C), whole),
        ],
        out_specs=pl.BlockSpec((_BN, C), row2),
        out_shape=jax.ShapeDtypeStruct((N, C), jnp.float32),
    )(q, u2, dis, b2.reshape(1, -1), Wl, bl.reshape(1, -1))

    return out


# speedup vs baseline: 1.0772x; 1.0772x over previous
import functools

import jax
import jax.numpy as jnp
from jax import lax
from jax.experimental import pallas as pl
from jax.experimental.pallas import tpu as pltpu
from jax.experimental.pallas import tpu_sc as plsc

N = 10000
F = 128
NC, NS = 2, 16
NW = NC * NS
CHUNK = 128
NPAD = N + 112
DEGP = 10240
FH = F // 2


def _tile_base(c, s, cpt0, cpt1):
    return jnp.where(c == 0, s * cpt0, NS * cpt0 + s * cpt1)


def _deg_body(dstc_hbm, zeros_hbm, out_hbm, dst_v, ones_v, acc_sh, cpt0, cpt1):
    c = lax.axis_index("c")
    s = lax.axis_index("s")
    seg = DEGP // NS
    base = _tile_base(c, s, cpt0, cpt1)
    cptc = jnp.where(c == 0, cpt0, cpt1)
    pltpu.sync_copy(zeros_hbm.at[pl.ds(s * seg, seg)],
                    acc_sh.at[pl.ds(s * seg, seg)])
    pltpu.sync_copy(dstc_hbm.at[pl.ds(base, max(cpt0, cpt1))], dst_v)
    for i in range(CHUNK // 16):
        ones_v[pl.ds(i * 16, 16)] = jnp.ones((16,), jnp.float32)
    plsc.subcore_barrier()

    @pl.loop(0, cptc)
    def _(j):
        pltpu.sync_copy(ones_v, acc_sh.at[dst_v.at[j]], add=True)

    plsc.subcore_barrier()
    pltpu.sync_copy(acc_sh.at[pl.ds(s * seg, seg)],
                    out_hbm.at[c, pl.ds(s * seg, seg)])


GRP = 8
NBUF = 8
IDXR = 3


def _agg_body(u_hbm, srcc_hbm, dstc_hbm, zeros_hbm, out_hbm,
              sidx, didx, rows_v, acc_sh, sl, sgs, sss, cpt, srows):
    c = lax.axis_index("c")
    s = lax.axis_index("s")
    seg = NPAD // NS
    base = c * srows + s * cpt
    ngroups = cpt // GRP
    pltpu.sync_copy(zeros_hbm.at[pl.ds(s * seg, seg)],
                    acc_sh.at[pl.ds(s * seg, seg)])
    pltpu.sync_copy(srcc_hbm.at[pl.ds(base, GRP)], sidx.at[0])
    pltpu.sync_copy(dstc_hbm.at[pl.ds(base, GRP)], didx.at[0])
    pltpu.sync_copy(srcc_hbm.at[pl.ds(base + GRP, GRP)], sidx.at[1])
    pltpu.sync_copy(dstc_hbm.at[pl.ds(base + GRP, GRP)], didx.at[1])
    plsc.subcore_barrier()

    wtg = tuple(pltpu.make_async_copy(u_hbm.at[pl.ds(0, CHUNK)],
                                      rows_v.at[b], sgs[b])
                for b in range(NBUF))
    wts = tuple(pltpu.make_async_copy(rows_v.at[b],
                                      acc_sh.at[didx.at[0, 0]], sss[b])
                for b in range(NBUF))
    wtl = (pltpu.make_async_copy(srcc_hbm.at[pl.ds(0, GRP)], sidx.at[0], sl),
           pltpu.make_async_copy(dstc_hbm.at[pl.ds(0, GRP)], didx.at[0], sl))

    def gath(slot, r, b):
        pltpu.async_copy(u_hbm.at[sidx.at[slot, r]], rows_v.at[b], sgs[b])

    def scat(slot, r, b):
        pltpu.async_copy(rows_v.at[b], acc_sh.at[didx.at[slot, r]], sss[b],
                         add=True)

    for b in range(NBUF - 1):
        gath(0, b, b)

    def do_group(g, slot, slot1, slotp, first):
        for r in range(GRP):
            wtg[r].wait()
            scat(slot, r, r)
            if not (first and r == 0):
                wts[(r + 7) % NBUF].wait()
            if r == 0:
                nb = base + (g + 2) * GRP
                pltpu.async_copy(srcc_hbm.at[pl.ds(nb, GRP)], sidx.at[slotp],
                                 sl)
                pltpu.async_copy(dstc_hbm.at[pl.ds(nb, GRP)], didx.at[slotp],
                                 sl)
            if r == 0:
                gath(slot, 7, 7)
            else:
                gath(slot1, r - 1, r - 1)
            if r == GRP - 1:
                wtl[0].wait()
                wtl[1].wait()

    do_group(0, 0, 1, 2, True)

    @pl.loop(1, ngroups)
    def _(g):
        slot = lax.rem(g, IDXR)
        slot1 = lax.rem(g + 1, IDXR)
        slotp = lax.rem(g + 2, IDXR)
        do_group(g, slot, slot1, slotp, False)

    wts[NBUF - 1].wait()
    for b in range(NBUF - 1):
        wtg[b].wait()
    plsc.subcore_barrier()
    pltpu.sync_copy(acc_sh.at[pl.ds(s * seg, seg)],
                    out_hbm.at[c, pl.ds(s * seg, seg)])


def _sc_mesh():
    return plsc.VectorSubcoreMesh(core_axis_name="c", subcore_axis_name="s",
                                  num_cores=NC, num_subcores=NS)


def _deg_call(dstc, zeros1, cpt0, cpt1):
    return pl.kernel(
        functools.partial(_deg_body, cpt0=cpt0, cpt1=cpt1),
        out_type=jax.ShapeDtypeStruct((NC, DEGP), jnp.float32),
        mesh=_sc_mesh(),
        scratch_types=[
            pltpu.VMEM((max(cpt0, cpt1), CHUNK), jnp.int32),
            pltpu.VMEM((CHUNK,), jnp.float32),
            pltpu.VMEM_SHARED((DEGP,), jnp.float32),
        ],
    )(dstc, zeros1)


def _agg_call(u, srcc, dstc, zeros2, cpt, srows):
    return pl.kernel(
        functools.partial(_agg_body, cpt=cpt, srows=srows),
        out_type=jax.ShapeDtypeStruct((NC, NPAD, FH), jnp.float32),
        mesh=_sc_mesh(),
        compiler_params=pltpu.CompilerParams(use_tc_tiling_on_sc=False),
        scratch_types=[
            pltpu.VMEM((IDXR, GRP, CHUNK), jnp.int32),
            pltpu.VMEM((IDXR, GRP, CHUNK), jnp.int32),
            pltpu.VMEM((NBUF, CHUNK, FH), jnp.float32),
            pltpu.VMEM_SHARED((NPAD, FH), jnp.float32),
            pltpu.SemaphoreType.DMA,
            [pltpu.SemaphoreType.DMA] * NBUF,
            [pltpu.SemaphoreType.DMA] * NBUF,
        ],
    )(u, srcc, dstc, zeros2)


def _tc1_body(x_ref, degp_ref, u1_ref, dis_ref):
    deg = degp_ref[0] + degp_ref[1] + 1.0
    dis = lax.rsqrt(deg)
    dis_ref[...] = dis
    xw = x_ref[...] * dis
    u1_ref[0] = xw[:, :FH]
    u1_ref[1] = xw[:, FH:]


def _tc2_body(p_ref, u1_ref, dis_ref, w1_ref, b1_ref, w2_ref, u2_ref):
    dis = dis_ref[...]
    agg1 = jnp.concatenate(
        [p_ref[0] + u1_ref[0], p_ref[1] + u1_ref[1]], axis=-1) * dis
    h1 = jax.nn.relu(
        jnp.dot(agg1, w1_ref[...], preferred_element_type=jnp.float32)
        + b1_ref[...])
    g2 = jnp.dot(h1, w2_ref[...], preferred_element_type=jnp.float32)
    u2 = g2 * dis
    u2_ref[0] = u2[:, :FH]
    u2_ref[1] = u2[:, FH:]


def _tc3_body(q_ref, u2_ref, dis_ref, b2_ref, wl_ref, bl_ref, out_ref):
    agg2 = jnp.concatenate(
        [q_ref[0] + u2_ref[0], q_ref[1] + u2_ref[1]], axis=-1) * dis_ref[...]
    h2 = jax.nn.relu(agg2 + b2_ref[...])
    lg = jnp.dot(h2, wl_ref[...], preferred_element_type=jnp.float32) + bl_ref[...]
    m = jnp.max(lg, axis=-1, keepdims=True)
    lse = m + jnp.log(jnp.sum(jnp.exp(lg - m), axis=-1, keepdims=True))
    out_ref[...] = lg - lse


_BN = 2000


def kernel(x, edge_index, W1, b1, W2, b2, Wl, bl):
    E = edge_index.shape[1]
    cpt = -(-E // (NS * CHUNK))
    cpt = -(-cpt // 16) * 16
    cpt0d = cpt * 4 // 5 // 8 * 8
    cpt1d = cpt - cpt0d
    tot = NS * cpt * CHUNK
    src = edge_index[0]
    dst = edge_index[1]
    pad = tot - E
    padrows = max(cpt0d - cpt1d, 16)
    srows = NS * cpt + padrows
    idxpad = jnp.zeros((pad + padrows * CHUNK,), jnp.int32)
    srcc = jnp.concatenate(
        [src, idxpad, src + NPAD, idxpad]).reshape(2 * srows, CHUNK)
    dpad = jnp.concatenate(
        [jnp.full((pad,), N, jnp.int32),
         jnp.zeros((padrows * CHUNK,), jnp.int32)])
    dstc = jnp.concatenate([dst, dpad, dst, dpad]).reshape(2 * srows, CHUNK)
    zeros1 = jnp.zeros((DEGP,), jnp.float32)
    zeros2 = jnp.zeros((NPAD, FH), jnp.float32)

    degp = _deg_call(dstc, zeros1, cpt0d, cpt1d)
    degp2 = degp[:, :, None]

    grid = (N // _BN,)
    row3 = lambda i: (0, i, 0)
    row2 = lambda i: (i, 0)
    whole = lambda i: (0, 0)

    u1, dis = pl.pallas_call(
        _tc1_body,
        grid=grid,
        in_specs=[
            pl.BlockSpec((_BN, F), row2),
            pl.BlockSpec((NC, _BN, 1), row3),
        ],
        out_specs=[
            pl.BlockSpec((NC, _BN, FH), row3),
            pl.BlockSpec((_BN, 1), row2),
        ],
        out_shape=[
            jax.ShapeDtypeStruct((NC, NPAD, FH), jnp.float32),
            jax.ShapeDtypeStruct((N, 1), jnp.float32),
        ],
    )(x, degp2)

    p = _agg_call(u1.reshape(NC * NPAD, FH), srcc, dstc, zeros2, cpt, srows)

    u2 = pl.pallas_call(
        _tc2_body,
        grid=grid,
        in_specs=[
            pl.BlockSpec((NC, _BN, FH), row3),
            pl.BlockSpec((NC, _BN, FH), row3),
            pl.BlockSpec((_BN, 1), row2),
            pl.BlockSpec((F, 2 * F), whole),
            pl.BlockSpec((1, 2 * F), whole),
            pl.BlockSpec((2 * F, F), whole),
        ],
        out_specs=pl.BlockSpec((NC, _BN, FH), row3),
        out_shape=jax.ShapeDtypeStruct((NC, NPAD, FH), jnp.float32),
    )(p, u1, dis, W1, b1.reshape(1, -1), W2)

    q = _agg_call(u2.reshape(NC * NPAD, FH), srcc, dstc, zeros2, cpt, srows)

    C = Wl.shape[1]
    out = pl.pallas_call(
        _tc3_body,
        grid=grid,
        in_specs=[
            pl.BlockSpec((NC, _BN, FH), row3),
            pl.BlockSpec((NC, _BN, FH), row3),
            pl.BlockSpec((_BN, 1), row2),
            pl.BlockSpec((1, F), whole),
            pl.BlockSpec((F, C), whole),
            pl.BlockSpec((1, C), whole),
        ],
        out_specs=pl.BlockSpec((_BN, C), row2),
        out_shape=jax.ShapeDtypeStruct((N, C), jnp.float32),
    )(q, u2, dis, b2.reshape(1, -1), Wl, bl.reshape(1, -1))

    return out


# speedup vs baseline: 1.1071x; 1.0278x over previous
import functools

import jax
import jax.numpy as jnp
from jax import lax
from jax.experimental import pallas as pl
from jax.experimental.pallas import tpu as pltpu
from jax.experimental.pallas import tpu_sc as plsc

N = 10000
F = 128
NC, NS = 2, 16
NW = NC * NS
CHUNK = 128
NPAD = N + 112
DEGP = 10240
FH = F // 2


def _deg_body(dstc_hbm, zeros_hbm, out_hbm, dst_v, ones_v, acc_sh, cpt, srows):
    c = lax.axis_index("c")
    s = lax.axis_index("s")
    seg = DEGP // NS
    base = c * srows + s * cpt
    pltpu.sync_copy(zeros_hbm.at[pl.ds(s * seg, seg)],
                    acc_sh.at[pl.ds(s * seg, seg)])
    pltpu.sync_copy(dstc_hbm.at[pl.ds(base, cpt)], dst_v)
    for i in range(CHUNK // 16):
        ones_v[pl.ds(i * 16, 16)] = jnp.ones((16,), jnp.float32)
    plsc.subcore_barrier()

    @pl.loop(0, cpt)
    def _(j):
        pltpu.sync_copy(ones_v, acc_sh.at[dst_v.at[j]], add=True)

    plsc.subcore_barrier()
    pltpu.sync_copy(acc_sh.at[pl.ds(s * seg, seg)],
                    out_hbm.at[c, pl.ds(s * seg, seg)])


GRP = 8
NBUF = 2
IDXR = 3


def _agg_body(u_hbm, srcc_hbm, dstc_hbm, zeros_hbm, out_hbm,
              sidx, didx, rows_v, acc_sh, sl, sgs, sss, cpt, srows):
    c = lax.axis_index("c")
    s = lax.axis_index("s")
    seg = NPAD // NS
    base = c * srows + s * cpt
    ngroups = cpt // GRP
    pltpu.sync_copy(zeros_hbm.at[pl.ds(s * seg, seg)],
                    acc_sh.at[pl.ds(s * seg, seg)])
    pltpu.sync_copy(srcc_hbm.at[pl.ds(base, GRP)], sidx.at[0])
    pltpu.sync_copy(dstc_hbm.at[pl.ds(base, GRP)], didx.at[0])
    pltpu.sync_copy(srcc_hbm.at[pl.ds(base + GRP, GRP)], sidx.at[1])
    pltpu.sync_copy(dstc_hbm.at[pl.ds(base + GRP, GRP)], didx.at[1])
    plsc.subcore_barrier()

    wtg = tuple(pltpu.make_async_copy(u_hbm.at[pl.ds(0, CHUNK)],
                                      rows_v.at[b], sgs[b])
                for b in range(NBUF))
    wts = tuple(pltpu.make_async_copy(rows_v.at[b],
                                      acc_sh.at[didx.at[0, 0]], sss[b])
                for b in range(NBUF))
    wtl = (pltpu.make_async_copy(srcc_hbm.at[pl.ds(0, GRP)], sidx.at[0], sl),
           pltpu.make_async_copy(dstc_hbm.at[pl.ds(0, GRP)], didx.at[0], sl))

    def gath(slot, r, b):
        pltpu.async_copy(u_hbm.at[sidx.at[slot, r]], rows_v.at[b], sgs[b])

    def scat(slot, r, b):
        pltpu.async_copy(rows_v.at[b], acc_sh.at[didx.at[slot, r]], sss[b],
                         add=True)

    for b in range(NBUF - 1):
        gath(0, b, b)

    def do_group(g, slot, slot1, slotp, first):
        for r in range(GRP):
            wtg[r % NBUF].wait()
            scat(slot, r, r % NBUF)
            if not (first and r == 0):
                wts[(r + NBUF - 1) % NBUF].wait()
            if r == 0:
                nb = base + (g + 2) * GRP
                pltpu.async_copy(srcc_hbm.at[pl.ds(nb, GRP)], sidx.at[slotp],
                                 sl)
                pltpu.async_copy(dstc_hbm.at[pl.ds(nb, GRP)], didx.at[slotp],
                                 sl)
            ahead = r + NBUF - 1
            if ahead < GRP:
                gath(slot, ahead, ahead % NBUF)
            else:
                gath(slot1, ahead - GRP, ahead % NBUF)
            if r == GRP - 1:
                wtl[0].wait()
                wtl[1].wait()

    do_group(0, 0, 1, 2, True)

    @pl.loop(1, ngroups)
    def _(g):
        slot = lax.rem(g, IDXR)
        slot1 = lax.rem(g + 1, IDXR)
        slotp = lax.rem(g + 2, IDXR)
        do_group(g, slot, slot1, slotp, False)

    wts[NBUF - 1].wait()
    for b in range(NBUF - 1):
        wtg[b].wait()
    plsc.subcore_barrier()
    pltpu.sync_copy(acc_sh.at[pl.ds(s * seg, seg)],
                    out_hbm.at[c, pl.ds(s * seg, seg)])


def _sc_mesh():
    return plsc.VectorSubcoreMesh(core_axis_name="c", subcore_axis_name="s",
                                  num_cores=NC, num_subcores=NS)


def _deg_call(dstc, zeros1, cpt, srows):
    return pl.kernel(
        functools.partial(_deg_body, cpt=cpt, srows=srows),
        out_type=jax.ShapeDtypeStruct((NC, DEGP), jnp.float32),
        mesh=_sc_mesh(),
        scratch_types=[
            pltpu.VMEM((cpt, CHUNK), jnp.int32),
            pltpu.VMEM((CHUNK,), jnp.float32),
            pltpu.VMEM_SHARED((DEGP,), jnp.float32),
        ],
    )(dstc, zeros1)


def _agg_call(u, srcc, dstc, zeros2, cpt, srows):
    return pl.kernel(
        functools.partial(_agg_body, cpt=cpt, srows=srows),
        out_type=jax.ShapeDtypeStruct((NC, NPAD, F), jnp.float32),
        mesh=_sc_mesh(),
        compiler_params=pltpu.CompilerParams(use_tc_tiling_on_sc=False),
        scratch_types=[
            pltpu.VMEM((IDXR, GRP, CHUNK), jnp.int32),
            pltpu.VMEM((IDXR, GRP, CHUNK), jnp.int32),
            pltpu.VMEM((NBUF, CHUNK, F), jnp.float32),
            pltpu.VMEM_SHARED((NPAD, F), jnp.float32),
            pltpu.SemaphoreType.DMA,
            [pltpu.SemaphoreType.DMA] * NBUF,
            [pltpu.SemaphoreType.DMA] * NBUF,
        ],
    )(u, srcc, dstc, zeros2)


def _tc1_body(x_ref, degp_ref, u1_ref, dis_ref):
    deg = degp_ref[0] + degp_ref[1] + 1.0
    dis = lax.rsqrt(deg)
    dis_ref[...] = dis
    xw = x_ref[...] * dis
    u1_ref[0] = xw
    u1_ref[1] = xw


def _tc2_body(p_ref, u1_ref, dis_ref, w1_ref, b1_ref, w2_ref, u2_ref):
    dis = dis_ref[...]
    agg1 = (p_ref[0] + p_ref[1] + u1_ref[0]) * dis
    h1 = jax.nn.relu(
        jnp.dot(agg1, w1_ref[...], preferred_element_type=jnp.float32)
        + b1_ref[...])
    g2 = jnp.dot(h1, w2_ref[...], preferred_element_type=jnp.float32)
    u2 = g2 * dis
    u2_ref[0] = u2
    u2_ref[1] = u2


def _tc3_body(q_ref, u2_ref, dis_ref, b2_ref, wl_ref, bl_ref, out_ref):
    agg2 = (q_ref[0] + q_ref[1] + u2_ref[0]) * dis_ref[...]
    h2 = jax.nn.relu(agg2 + b2_ref[...])
    lg = jnp.dot(h2, wl_ref[...], preferred_element_type=jnp.float32) + bl_ref[...]
    m = jnp.max(lg, axis=-1, keepdims=True)
    lse = m + jnp.log(jnp.sum(jnp.exp(lg - m), axis=-1, keepdims=True))
    out_ref[...] = lg - lse


_BN = 2000


def kernel(x, edge_index, W1, b1, W2, b2, Wl, bl):
    E = edge_index.shape[1]
    n0 = E // 2
    n1 = E - n0
    cpt = -(-max(n0, n1) // (NS * CHUNK))
    cpt = -(-cpt // 8) * 8
    cap = NS * cpt * CHUNK
    src = edge_index[0]
    dst = edge_index[1]
    padrows = 16
    srows = NS * cpt + padrows
    sp0 = jnp.zeros(((cap - n0) + padrows * CHUNK,), jnp.int32)
    sp1 = jnp.zeros(((cap - n1) + padrows * CHUNK,), jnp.int32)
    srcc = jnp.concatenate(
        [src[:n0], sp0, src[n0:] + NPAD, sp1]).reshape(2 * srows, CHUNK)
    dp0 = jnp.concatenate(
        [jnp.full(((cap - n0),), N, jnp.int32),
         jnp.zeros((padrows * CHUNK,), jnp.int32)])
    dp1 = jnp.concatenate(
        [jnp.full(((cap - n1),), N, jnp.int32),
         jnp.zeros((padrows * CHUNK,), jnp.int32)])
    dstc = jnp.concatenate(
        [dst[:n0], dp0, dst[n0:], dp1]).reshape(2 * srows, CHUNK)
    zeros1 = jnp.zeros((DEGP,), jnp.float32)
    zeros2 = jnp.zeros((NPAD, F), jnp.float32)

    degp = _deg_call(dstc, zeros1, cpt, srows)
    degp2 = degp[:, :, None]

    grid = (N // _BN,)
    row3 = lambda i: (0, i, 0)
    row2 = lambda i: (i, 0)
    whole = lambda i: (0, 0)

    u1, dis = pl.pallas_call(
        _tc1_body,
        grid=grid,
        in_specs=[
            pl.BlockSpec((_BN, F), row2),
            pl.BlockSpec((NC, _BN, 1), row3),
        ],
        out_specs=[
            pl.BlockSpec((NC, _BN, F), row3),
            pl.BlockSpec((_BN, 1), row2),
        ],
        out_shape=[
            jax.ShapeDtypeStruct((NC, NPAD, F), jnp.float32),
            jax.ShapeDtypeStruct((N, 1), jnp.float32),
        ],
    )(x, degp2)

    p = _agg_call(u1.reshape(NC * NPAD, F), srcc, dstc, zeros2, cpt, srows)

    u2 = pl.pallas_call(
        _tc2_body,
        grid=grid,
        in_specs=[
            pl.BlockSpec((NC, _BN, F), row3),
            pl.BlockSpec((1, _BN, F), row3),
            pl.BlockSpec((_BN, 1), row2),
            pl.BlockSpec((F, 2 * F), whole),
            pl.BlockSpec((1, 2 * F), whole),
            pl.BlockSpec((2 * F, F), whole),
        ],
        out_specs=pl.BlockSpec((NC, _BN, F), row3),
        out_shape=jax.ShapeDtypeStruct((NC, NPAD, F), jnp.float32),
    )(p, u1, dis, W1, b1.reshape(1, -1), W2)

    q = _agg_call(u2.reshape(NC * NPAD, F), srcc, dstc, zeros2, cpt, srows)

    C = Wl.shape[1]
    out = pl.pallas_call(
        _tc3_body,
        grid=grid,
        in_specs=[
            pl.BlockSpec((NC, _BN, F), row3),
            pl.BlockSpec((1, _BN, F), row3),
            pl.BlockSpec((_BN, 1), row2),
            pl.BlockSpec((1, F), whole),
            pl.BlockSpec((F, C), whole),
            pl.BlockSpec((1, C), whole),
        ],
        out_specs=pl.BlockSpec((_BN, C), row2),
        out_shape=jax.ShapeDtypeStruct((N, C), jnp.float32),
    )(q, u2, dis, b2.reshape(1, -1), Wl, bl.reshape(1, -1))

    return out
